# baseline (speedup 1.0000x reference)
import jax
import jax.numpy as jnp
from jax.experimental import pallas as pl
from jax.experimental.pallas import tpu as pltpu

N = 10000
E = 160000
DT = 1.0
PACK = 8

GRID = 10
NODE_BLOCK = N // GRID
EDGE_BLOCK = E // PACK // GRID


def _fused_kernel(x_ref, w1_ref, b1_ref, w2_ref, b2_ref, wdn_ref, bdn_ref,
                  e_ref, wde_ref, bde_ref, on_ref, oe_ref):
    x = x_ref[...]
    h = jnp.dot(x, w1_ref[...], preferred_element_type=jnp.float32) + b1_ref[...]
    h = h * jax.nn.sigmoid(h)
    h = jnp.dot(h, w2_ref[...], preferred_element_type=jnp.float32) + b2_ref[...]
    h = h * jax.nn.sigmoid(h)
    on_ref[...] = jnp.dot(h, wdn_ref[...], preferred_element_type=jnp.float32) + bdn_ref[...]
    oe_ref[...] = jnp.dot(e_ref[...], wde_ref[...], preferred_element_type=jnp.float32) + bde_ref[...]


def kernel(nodes, edges, senders, receivers, globals_, W_enc_n, b_enc_n, W_enc_e, b_enc_e, W1, b1, W2, b2, Wd_n, bd_n, Wd_e, bd_e):
    d_feat = nodes.shape[1]
    latent = W1.shape[1]
    node_out = Wd_n.shape[1]
    d_edge = edges.shape[1]
    edge_out = Wd_e.shape[1]

    edges_p = edges.reshape(E // PACK, PACK * d_edge)
    Wde_bd = jnp.kron(jnp.eye(PACK, dtype=Wd_e.dtype), Wd_e)
    bde_t = jnp.tile(bd_e, PACK).reshape(1, -1)

    whole = lambda *shape: pl.BlockSpec(shape, lambda i: (0,) * len(shape))

    out_nodes, out_edges_p = pl.pallas_call(
        _fused_kernel,
        grid=(GRID,),
        in_specs=[
            pl.BlockSpec((NODE_BLOCK, d_feat), lambda i: (i, 0)),
            whole(d_feat, latent),
            whole(1, latent),
            whole(latent, latent),
            whole(1, latent),
            whole(latent, node_out),
            whole(1, node_out),
            pl.BlockSpec((EDGE_BLOCK, PACK * d_edge), lambda i: (i, 0)),
            whole(PACK * d_edge, PACK * edge_out),
            whole(1, PACK * edge_out),
        ],
        out_specs=[
            pl.BlockSpec((NODE_BLOCK, node_out), lambda i: (i, 0)),
            pl.BlockSpec((EDGE_BLOCK, PACK * edge_out), lambda i: (i, 0)),
        ],
        out_shape=[
            jax.ShapeDtypeStruct((N, node_out), jnp.float32),
            jax.ShapeDtypeStruct((E // PACK, PACK * edge_out), jnp.float32),
        ],
        compiler_params=pltpu.CompilerParams(
            dimension_semantics=("parallel",),
        ),
    )(nodes, W1, b1.reshape(1, -1), W2, b2.reshape(1, -1), Wd_n, bd_n.reshape(1, -1),
      edges_p, Wde_bd, bde_t)

    out_edges = out_edges_p.reshape(E, edge_out)
    new_globals = globals_ + DT
    return out_nodes, out_edges, new_globals
